# Initial kernel scaffold; baseline (speedup 1.0000x reference)
import jax
import jax.numpy as jnp
from jax.experimental import pallas as pl


def kernel(x_cont, x_month, x_weekday, x_dir, W_proj, b_proj, E_month, E_weekday, E_dir):
    raise NotImplementedError("write your pallas kernel here")


# speedup vs baseline: 4.8254x; 4.8254x over previous
import math

import jax
import jax.numpy as jnp
import numpy as np
from jax.experimental import pallas as pl
from jax.experimental.pallas import tpu as pltpu

D_MODEL = 128
L_SEQ = 50
BM = 64
R = BM * L_SEQ


def _pe_const(d_model: int, max_len: int) -> np.ndarray:
    pos = np.arange(0, max_len, dtype=np.float32)[:, None]
    div = np.exp(
        np.arange(0, d_model, 2, dtype=np.float32) * (-math.log(10000.0) / d_model)
    )
    pe = np.zeros((max_len, d_model), dtype=np.float32)
    pe[:, 0::2] = np.sin(pos * div)
    pe[:, 1::2] = np.cos(pos * div)
    return pe


_PE50 = _pe_const(D_MODEL, L_SEQ)


def _fused_kernel(xc_ref, idx_ref, wp_ref, et_ref, peb_ref, out_ref):
    xc = xc_ref[...]
    idx = idx_ref[...]
    m = idx[:, 0:1]
    w = idx[:, 1:2]
    d = idx[:, 2:3]
    iota = jax.lax.broadcasted_iota(jnp.int32, (idx.shape[0], 27), 1)
    oh = ((iota == m) | (iota == w + 12) | (iota == d + 19)).astype(jnp.float32)
    acc = jnp.dot(xc, wp_ref[...], preferred_element_type=jnp.float32)
    acc = acc + jnp.dot(oh, et_ref[...], preferred_element_type=jnp.float32)
    out_ref[...] = acc + peb_ref[...]


def kernel(x_cont, x_month, x_weekday, x_dir, W_proj, b_proj, E_month, E_weekday, E_dir):
    B, L, C = x_cont.shape
    N = B * L
    xc2 = x_cont.reshape(N, C)
    idx3 = jnp.stack(
        [x_month.astype(jnp.int32), x_weekday.astype(jnp.int32), x_dir.astype(jnp.int32)],
        axis=-1,
    ).reshape(N, 3)
    e_all = jnp.concatenate([E_month, E_weekday, E_dir], axis=0)
    pe_block = jnp.tile(jnp.asarray(_PE50), (BM, 1)) + b_proj[None, :]

    grid = (N // R,)
    out2 = pl.pallas_call(
        _fused_kernel,
        grid=grid,
        in_specs=[
            pl.BlockSpec((R, C), lambda i: (i, 0)),
            pl.BlockSpec((R, 3), lambda i: (i, 0)),
            pl.BlockSpec((C, D_MODEL), lambda i: (0, 0)),
            pl.BlockSpec((27, D_MODEL), lambda i: (0, 0)),
            pl.BlockSpec((R, D_MODEL), lambda i: (0, 0)),
        ],
        out_specs=pl.BlockSpec((R, D_MODEL), lambda i: (i, 0)),
        out_shape=jax.ShapeDtypeStruct((N, D_MODEL), jnp.float32),
        compiler_params=pltpu.CompilerParams(
            dimension_semantics=("arbitrary",),
        ),
    )(xc2, idx3, W_proj, e_all, pe_block)
    return out2.reshape(B, L, D_MODEL)


# speedup vs baseline: 5.0102x; 1.0383x over previous
import math

import jax
import jax.numpy as jnp
import numpy as np
from jax.experimental import pallas as pl
from jax.experimental.pallas import tpu as pltpu

D_MODEL = 128
L_SEQ = 50
BM = 128
R = BM * L_SEQ


def _pe_const(d_model: int, max_len: int) -> np.ndarray:
    pos = np.arange(0, max_len, dtype=np.float32)[:, None]
    div = np.exp(
        np.arange(0, d_model, 2, dtype=np.float32) * (-math.log(10000.0) / d_model)
    )
    pe = np.zeros((max_len, d_model), dtype=np.float32)
    pe[:, 0::2] = np.sin(pos * div)
    pe[:, 1::2] = np.cos(pos * div)
    return pe


_PE50 = _pe_const(D_MODEL, L_SEQ)


def _fused_kernel(xc_ref, idx_ref, wp_ref, et_ref, peb_ref, out_ref):
    xc = xc_ref[...]
    idx = idx_ref[...]
    m = idx[:, 0:1]
    w = idx[:, 1:2]
    d = idx[:, 2:3]
    iota = jax.lax.broadcasted_iota(jnp.int32, (idx.shape[0], 27), 1)
    oh = ((iota == m) | (iota == w + 12) | (iota == d + 19)).astype(jnp.float32)
    acc = jnp.dot(xc, wp_ref[...], preferred_element_type=jnp.float32)
    acc = acc + jnp.dot(oh, et_ref[...], preferred_element_type=jnp.float32)
    out_ref[...] = acc + peb_ref[...]


def kernel(x_cont, x_month, x_weekday, x_dir, W_proj, b_proj, E_month, E_weekday, E_dir):
    B, L, C = x_cont.shape
    N = B * L
    xc2 = x_cont.reshape(N, C)
    idx3 = jnp.stack(
        [x_month.astype(jnp.int32), x_weekday.astype(jnp.int32), x_dir.astype(jnp.int32)],
        axis=-1,
    ).reshape(N, 3)
    e_all = jnp.concatenate([E_month, E_weekday, E_dir], axis=0)
    pe_block = jnp.tile(jnp.asarray(_PE50), (BM, 1)) + b_proj[None, :]

    grid = (N // R,)
    out2 = pl.pallas_call(
        _fused_kernel,
        grid=grid,
        in_specs=[
            pl.BlockSpec((R, C), lambda i: (i, 0)),
            pl.BlockSpec((R, 3), lambda i: (i, 0)),
            pl.BlockSpec((C, D_MODEL), lambda i: (0, 0)),
            pl.BlockSpec((27, D_MODEL), lambda i: (0, 0)),
            pl.BlockSpec((R, D_MODEL), lambda i: (0, 0)),
        ],
        out_specs=pl.BlockSpec((R, D_MODEL), lambda i: (i, 0)),
        out_shape=jax.ShapeDtypeStruct((N, D_MODEL), jnp.float32),
        compiler_params=pltpu.CompilerParams(
            dimension_semantics=("parallel",),
        ),
    )(xc2, idx3, W_proj, e_all, pe_block)
    return out2.reshape(B, L, D_MODEL)


# speedup vs baseline: 6.3499x; 1.2674x over previous
import math

import jax
import jax.numpy as jnp
import numpy as np
from jax.experimental import pallas as pl
from jax.experimental.pallas import tpu as pltpu

D_MODEL = 128
L_SEQ = 50
BM = 128
R = BM * L_SEQ


def _pe_const(d_model: int, max_len: int) -> np.ndarray:
    pos = np.arange(0, max_len, dtype=np.float32)[:, None]
    div = np.exp(
        np.arange(0, d_model, 2, dtype=np.float32) * (-math.log(10000.0) / d_model)
    )
    pe = np.zeros((max_len, d_model), dtype=np.float32)
    pe[:, 0::2] = np.sin(pos * div)
    pe[:, 1::2] = np.cos(pos * div)
    return pe


_PE50 = _pe_const(D_MODEL, L_SEQ)


def _fused_kernel(xc_ref, idx_ref, wp_ref, et_ref, peb_ref, out_ref):
    xc = xc_ref[...]
    idx = idx_ref[...]
    m = idx[:, 0:1]
    w = idx[:, 1:2]
    d = idx[:, 2:3]
    iota = jax.lax.broadcasted_iota(jnp.int32, (idx.shape[0], 27), 1)
    oh = ((iota == m) | (iota == w + 12) | (iota == d + 19)).astype(jnp.float32)
    acc = jnp.dot(xc, wp_ref[...], preferred_element_type=jnp.float32)
    acc = acc + jnp.dot(oh, et_ref[...], preferred_element_type=jnp.float32)
    acc = acc + peb_ref[...]
    out_ref[...] = acc.reshape(out_ref.shape)


def kernel(x_cont, x_month, x_weekday, x_dir, W_proj, b_proj, E_month, E_weekday, E_dir):
    B, L, C = x_cont.shape
    N = B * L
    xc2 = x_cont.reshape(N, C)
    idx3 = jnp.stack(
        [x_month.astype(jnp.int32), x_weekday.astype(jnp.int32), x_dir.astype(jnp.int32)],
        axis=-1,
    ).reshape(N, 3)
    e_all = jnp.concatenate([E_month, E_weekday, E_dir], axis=0)
    pe_block = jnp.tile(jnp.asarray(_PE50), (BM, 1)) + b_proj[None, :]

    grid = (N // R,)
    out2 = pl.pallas_call(
        _fused_kernel,
        grid=grid,
        in_specs=[
            pl.BlockSpec((R, C), lambda i: (i, 0)),
            pl.BlockSpec((R, 3), lambda i: (i, 0)),
            pl.BlockSpec((C, D_MODEL), lambda i: (0, 0)),
            pl.BlockSpec((27, D_MODEL), lambda i: (0, 0)),
            pl.BlockSpec((R, D_MODEL), lambda i: (0, 0)),
        ],
        out_specs=pl.BlockSpec((BM, L, D_MODEL), lambda i: (i, 0, 0)),
        out_shape=jax.ShapeDtypeStruct((B, L, D_MODEL), jnp.float32),
        compiler_params=pltpu.CompilerParams(
            dimension_semantics=("parallel",),
        ),
    )(xc2, idx3, W_proj, e_all, pe_block)
    return out2


# speedup vs baseline: 8.8452x; 1.3930x over previous
import math

import jax
import jax.numpy as jnp
import numpy as np
from jax.experimental import pallas as pl
from jax.experimental.pallas import tpu as pltpu

D_MODEL = 128
L_SEQ = 50
BM = 128
R = BM * L_SEQ


def _pe_const(d_model: int, max_len: int) -> np.ndarray:
    pos = np.arange(0, max_len, dtype=np.float32)[:, None]
    div = np.exp(
        np.arange(0, d_model, 2, dtype=np.float32) * (-math.log(10000.0) / d_model)
    )
    pe = np.zeros((max_len, d_model), dtype=np.float32)
    pe[:, 0::2] = np.sin(pos * div)
    pe[:, 1::2] = np.cos(pos * div)
    return pe


_PE50 = _pe_const(D_MODEL, L_SEQ)


def _fused_kernel(xc_ref, combo_ref, wp_ref, et_ref, peb_ref, out_ref):
    xc = xc_ref[...]
    combo = combo_ref[...].reshape(1, R)
    m = combo & 15
    w = (combo >> 4) & 15
    d = combo >> 8
    iota = jax.lax.broadcasted_iota(jnp.int32, (27, R), 0)
    oht = ((iota == m) | (iota == w + 12) | (iota == d + 19)).astype(jnp.float32)
    emb = jax.lax.dot_general(
        oht, et_ref[...], (((0,), (0,)), ((), ())),
        preferred_element_type=jnp.float32,
    )
    acc = jnp.dot(xc, wp_ref[...], preferred_element_type=jnp.float32)
    acc = acc + emb + peb_ref[...]
    out_ref[...] = acc.reshape(out_ref.shape)


def kernel(x_cont, x_month, x_weekday, x_dir, W_proj, b_proj, E_month, E_weekday, E_dir):
    B, L, C = x_cont.shape
    N = B * L
    xc2 = x_cont.reshape(N, C)
    combo = (
        x_month.astype(jnp.int32)
        | (x_weekday.astype(jnp.int32) << 4)
        | (x_dir.astype(jnp.int32) << 8)
    ).reshape(N // R, 1, R)
    e_all = jnp.concatenate([E_month, E_weekday, E_dir], axis=0)
    pe_block = jnp.tile(jnp.asarray(_PE50), (BM, 1)) + b_proj[None, :]

    grid = (N // R,)
    out2 = pl.pallas_call(
        _fused_kernel,
        grid=grid,
        in_specs=[
            pl.BlockSpec((R, C), lambda i: (i, 0)),
            pl.BlockSpec((1, 1, R), lambda i: (i, 0, 0)),
            pl.BlockSpec((C, D_MODEL), lambda i: (0, 0)),
            pl.BlockSpec((27, D_MODEL), lambda i: (0, 0)),
            pl.BlockSpec((R, D_MODEL), lambda i: (0, 0)),
        ],
        out_specs=pl.BlockSpec((BM, L, D_MODEL), lambda i: (i, 0, 0)),
        out_shape=jax.ShapeDtypeStruct((B, L, D_MODEL), jnp.float32),
        compiler_params=pltpu.CompilerParams(
            dimension_semantics=("parallel",),
        ),
    )(xc2, combo, W_proj, e_all, pe_block)
    return out2


# speedup vs baseline: 11.4262x; 1.2918x over previous
import math

import jax
import jax.numpy as jnp
import numpy as np
from jax.experimental import pallas as pl
from jax.experimental.pallas import tpu as pltpu

D_MODEL = 128
L_SEQ = 50
BM = 128
R = BM * L_SEQ


def _pe_const(d_model: int, max_len: int) -> np.ndarray:
    pos = np.arange(0, max_len, dtype=np.float32)[:, None]
    div = np.exp(
        np.arange(0, d_model, 2, dtype=np.float32) * (-math.log(10000.0) / d_model)
    )
    pe = np.zeros((max_len, d_model), dtype=np.float32)
    pe[:, 0::2] = np.sin(pos * div)
    pe[:, 1::2] = np.cos(pos * div)
    return pe


_PE50 = _pe_const(D_MODEL, L_SEQ)


def _fused_kernel(xc_ref, combo_ref, wp_ref, et_ref, peb_ref, out_ref):
    xct = xc_ref[...]
    combo = combo_ref[...].reshape(1, R)
    m = combo & 15
    w = (combo >> 4) & 15
    d = combo >> 8
    iota = jax.lax.broadcasted_iota(jnp.int32, (27, R), 0)
    oht = ((iota == m) | (iota == w + 12) | (iota == d + 19)).astype(jnp.float32)
    emb = jax.lax.dot_general(
        oht, et_ref[...], (((0,), (0,)), ((), ())),
        preferred_element_type=jnp.float32,
    )
    acc = jax.lax.dot_general(
        xct, wp_ref[...], (((0,), (0,)), ((), ())),
        preferred_element_type=jnp.float32,
    )
    acc = acc + emb + peb_ref[...]
    out_ref[...] = acc.reshape(out_ref.shape)


def kernel(x_cont, x_month, x_weekday, x_dir, W_proj, b_proj, E_month, E_weekday, E_dir):
    B, L, C = x_cont.shape
    N = B * L
    xc2 = x_cont.transpose(2, 0, 1).reshape(C, N)
    combo = (
        x_month.astype(jnp.int32)
        | (x_weekday.astype(jnp.int32) << 4)
        | (x_dir.astype(jnp.int32) << 8)
    ).reshape(N // R, 1, R)
    e_all = jnp.concatenate([E_month, E_weekday, E_dir], axis=0)
    pe_block = jnp.tile(jnp.asarray(_PE50), (BM, 1)) + b_proj[None, :]

    grid = (N // R,)
    out2 = pl.pallas_call(
        _fused_kernel,
        grid=grid,
        in_specs=[
            pl.BlockSpec((C, R), lambda i: (0, i)),
            pl.BlockSpec((1, 1, R), lambda i: (i, 0, 0)),
            pl.BlockSpec((C, D_MODEL), lambda i: (0, 0)),
            pl.BlockSpec((27, D_MODEL), lambda i: (0, 0)),
            pl.BlockSpec((R, D_MODEL), lambda i: (0, 0)),
        ],
        out_specs=pl.BlockSpec((BM, L, D_MODEL), lambda i: (i, 0, 0)),
        out_shape=jax.ShapeDtypeStruct((B, L, D_MODEL), jnp.float32),
        compiler_params=pltpu.CompilerParams(
            dimension_semantics=("parallel",),
        ),
    )(xc2, combo, W_proj, e_all, pe_block)
    return out2


# speedup vs baseline: 14.6574x; 1.2828x over previous
import math

import jax
import jax.numpy as jnp
import numpy as np
from jax.experimental import pallas as pl
from jax.experimental.pallas import tpu as pltpu

D_MODEL = 128
L_SEQ = 50
BM = 128
R = BM * L_SEQ


def _pe_const(d_model: int, max_len: int) -> np.ndarray:
    pos = np.arange(0, max_len, dtype=np.float32)[:, None]
    div = np.exp(
        np.arange(0, d_model, 2, dtype=np.float32) * (-math.log(10000.0) / d_model)
    )
    pe = np.zeros((max_len, d_model), dtype=np.float32)
    pe[:, 0::2] = np.sin(pos * div)
    pe[:, 1::2] = np.cos(pos * div)
    return pe


_PE50 = _pe_const(D_MODEL, L_SEQ)


def _fused_kernel(xc_ref, combo_ref, wp_ref, et_ref, peb_ref, out_ref):
    xct = xc_ref[...]
    combo = combo_ref[...].reshape(1, R)
    m = combo & 15
    w = (combo >> 4) & 15
    d = combo >> 8
    iota = jax.lax.broadcasted_iota(jnp.int32, (27, R), 0)
    oht = ((iota == m) | (iota == w + 12) | (iota == d + 19)).astype(jnp.float32)
    emb = jax.lax.dot_general(
        oht, et_ref[...], (((0,), (0,)), ((), ())),
        preferred_element_type=jnp.float32,
    )
    proj = jax.lax.dot_general(
        xct, wp_ref[...], (((0,), (0,)), ((), ())),
        preferred_element_type=jnp.float32,
    )
    acc = emb + peb_ref[...]
    out_ref[...] = proj + acc.reshape(out_ref.shape)


def kernel(x_cont, x_month, x_weekday, x_dir, W_proj, b_proj, E_month, E_weekday, E_dir):
    B, L, C = x_cont.shape
    N = B * L
    xc2 = x_cont.transpose(2, 0, 1)
    combo = (
        x_month.astype(jnp.int32)
        | (x_weekday.astype(jnp.int32) << 4)
        | (x_dir.astype(jnp.int32) << 8)
    ).reshape(N // R, 1, R)
    e_all = jnp.concatenate([E_month, E_weekday, E_dir], axis=0)
    pe_block = jnp.tile(jnp.asarray(_PE50), (BM, 1)) + b_proj[None, :]

    grid = (N // R,)
    out2 = pl.pallas_call(
        _fused_kernel,
        grid=grid,
        in_specs=[
            pl.BlockSpec((C, BM, L_SEQ), lambda i: (0, i, 0)),
            pl.BlockSpec((1, 1, R), lambda i: (i, 0, 0)),
            pl.BlockSpec((C, D_MODEL), lambda i: (0, 0)),
            pl.BlockSpec((27, D_MODEL), lambda i: (0, 0)),
            pl.BlockSpec((R, D_MODEL), lambda i: (0, 0)),
        ],
        out_specs=pl.BlockSpec((BM, L, D_MODEL), lambda i: (i, 0, 0)),
        out_shape=jax.ShapeDtypeStruct((B, L, D_MODEL), jnp.float32),
        compiler_params=pltpu.CompilerParams(
            dimension_semantics=("parallel",),
        ),
    )(xc2, combo, W_proj, e_all, pe_block)
    return out2


# speedup vs baseline: 19.8230x; 1.3524x over previous
import math

import jax
import jax.numpy as jnp
import numpy as np
from jax.experimental import pallas as pl
from jax.experimental.pallas import tpu as pltpu

D_MODEL = 128
L_SEQ = 50
BM = 128
R = BM * L_SEQ


def _pe_const(d_model: int, max_len: int) -> np.ndarray:
    pos = np.arange(0, max_len, dtype=np.float32)[:, None]
    div = np.exp(
        np.arange(0, d_model, 2, dtype=np.float32) * (-math.log(10000.0) / d_model)
    )
    pe = np.zeros((max_len, d_model), dtype=np.float32)
    pe[:, 0::2] = np.sin(pos * div)
    pe[:, 1::2] = np.cos(pos * div)
    return pe


_PE50 = _pe_const(D_MODEL, L_SEQ)


def _fused_kernel(xc_ref, combo_ref, wp_ref, et_ref, peb_ref, out_ref):
    xct = xc_ref[...]
    combo = combo_ref[...].reshape(1, R)
    m = combo & 15
    w = (combo >> 4) & 15
    d = combo >> 8
    iota = jax.lax.broadcasted_iota(jnp.int32, (27, R), 0)
    oht = ((iota == m) | (iota == w + 12) | (iota == d + 19)).astype(jnp.float32)
    emb = jax.lax.dot_general(
        oht, et_ref[...], (((0,), (0,)), ((), ())),
        preferred_element_type=jnp.float32,
    )
    proj = jax.lax.dot_general(
        xct, wp_ref[...], (((0,), (0,)), ((), ())),
        preferred_element_type=jnp.float32,
    )
    acc = emb + peb_ref[...]
    res = proj + acc.reshape(BM, L_SEQ, D_MODEL)
    out_ref[...] = res.transpose(1, 0, 2)


def kernel(x_cont, x_month, x_weekday, x_dir, W_proj, b_proj, E_month, E_weekday, E_dir):
    B, L, C = x_cont.shape
    N = B * L
    xc2 = x_cont.transpose(2, 0, 1)
    combo = (
        x_month.astype(jnp.int32)
        | (x_weekday.astype(jnp.int32) << 4)
        | (x_dir.astype(jnp.int32) << 8)
    ).reshape(N // R, 1, R)
    e_all = jnp.concatenate([E_month, E_weekday, E_dir], axis=0)
    pe_block = jnp.tile(jnp.asarray(_PE50), (BM, 1)) + b_proj[None, :]

    grid = (N // R,)
    out2 = pl.pallas_call(
        _fused_kernel,
        grid=grid,
        in_specs=[
            pl.BlockSpec((C, BM, L_SEQ), lambda i: (0, i, 0)),
            pl.BlockSpec((1, 1, R), lambda i: (i, 0, 0)),
            pl.BlockSpec((C, D_MODEL), lambda i: (0, 0)),
            pl.BlockSpec((27, D_MODEL), lambda i: (0, 0)),
            pl.BlockSpec((R, D_MODEL), lambda i: (0, 0)),
        ],
        out_specs=pl.BlockSpec((L, BM, D_MODEL), lambda i: (0, i, 0)),
        out_shape=jax.ShapeDtypeStruct((L, B, D_MODEL), jnp.float32),
        compiler_params=pltpu.CompilerParams(
            dimension_semantics=("parallel",),
        ),
    )(xc2, combo, W_proj, e_all, pe_block)
    return out2.transpose(1, 0, 2)


# speedup vs baseline: 35.9669x; 1.8144x over previous
import math

import jax
import jax.numpy as jnp
import numpy as np
from jax.experimental import pallas as pl
from jax.experimental.pallas import tpu as pltpu

D_MODEL = 128
L_SEQ = 50
BM = 128
R = BM * L_SEQ


def _pe_const(d_model: int, max_len: int) -> np.ndarray:
    pos = np.arange(0, max_len, dtype=np.float32)[:, None]
    div = np.exp(
        np.arange(0, d_model, 2, dtype=np.float32) * (-math.log(10000.0) / d_model)
    )
    pe = np.zeros((max_len, d_model), dtype=np.float32)
    pe[:, 0::2] = np.sin(pos * div)
    pe[:, 1::2] = np.cos(pos * div)
    return pe


_PE50 = _pe_const(D_MODEL, L_SEQ)


def _fused_kernel(xc_ref, combo_ref, wp_ref, et_ref, peb_ref, out_ref):
    xct = xc_ref[...]
    combo = combo_ref[...][None]
    m = combo & 15
    w = (combo >> 4) & 15
    d = combo >> 8
    iota = jax.lax.broadcasted_iota(jnp.int32, (27, L_SEQ, BM), 0)
    oht = ((iota == m) | (iota == w + 12) | (iota == d + 19)).astype(jnp.float32)
    emb = jax.lax.dot_general(
        oht, et_ref[...], (((0,), (0,)), ((), ())),
        preferred_element_type=jnp.float32,
    )
    proj = jax.lax.dot_general(
        xct, wp_ref[...], (((0,), (0,)), ((), ())),
        preferred_element_type=jnp.float32,
    )
    out_ref[...] = proj + emb + peb_ref[...]


def kernel(x_cont, x_month, x_weekday, x_dir, W_proj, b_proj, E_month, E_weekday, E_dir):
    B, L, C = x_cont.shape
    xc2 = x_cont.transpose(2, 1, 0)
    combo = (
        x_month.astype(jnp.int32)
        | (x_weekday.astype(jnp.int32) << 4)
        | (x_dir.astype(jnp.int32) << 8)
    ).T
    e_all = jnp.concatenate([E_month, E_weekday, E_dir], axis=0)
    pe_block = jnp.asarray(_PE50)[:, None, :] + b_proj[None, None, :]

    grid = (B // BM,)
    out2 = pl.pallas_call(
        _fused_kernel,
        grid=grid,
        in_specs=[
            pl.BlockSpec((C, L_SEQ, BM), lambda i: (0, 0, i)),
            pl.BlockSpec((L_SEQ, BM), lambda i: (0, i)),
            pl.BlockSpec((C, D_MODEL), lambda i: (0, 0)),
            pl.BlockSpec((27, D_MODEL), lambda i: (0, 0)),
            pl.BlockSpec((L_SEQ, 1, D_MODEL), lambda i: (0, 0, 0)),
        ],
        out_specs=pl.BlockSpec((L_SEQ, BM, D_MODEL), lambda i: (0, i, 0)),
        out_shape=jax.ShapeDtypeStruct((L, B, D_MODEL), jnp.float32),
        compiler_params=pltpu.CompilerParams(
            dimension_semantics=("parallel",),
        ),
    )(xc2, combo, W_proj, e_all, pe_block)
    return out2.transpose(1, 0, 2)


# speedup vs baseline: 44.9201x; 1.2489x over previous
import math

import jax
import jax.numpy as jnp
import numpy as np
from jax.experimental import pallas as pl
from jax.experimental.pallas import tpu as pltpu

D_MODEL = 128
L_SEQ = 50
BM = 128
R = BM * L_SEQ


def _pe_const(d_model: int, max_len: int) -> np.ndarray:
    pos = np.arange(0, max_len, dtype=np.float32)[:, None]
    div = np.exp(
        np.arange(0, d_model, 2, dtype=np.float32) * (-math.log(10000.0) / d_model)
    )
    pe = np.zeros((max_len, d_model), dtype=np.float32)
    pe[:, 0::2] = np.sin(pos * div)
    pe[:, 1::2] = np.cos(pos * div)
    return pe


_PE50 = _pe_const(D_MODEL, L_SEQ)


def _fused_kernel(xc_ref, combo_ref, wall_ref, peb_ref, out_ref):
    xct = xc_ref[...]
    combo = combo_ref[...][None]
    m = combo & 15
    w = (combo >> 4) & 15
    d = combo >> 8
    iota = jax.lax.broadcasted_iota(jnp.int32, (27, L_SEQ, BM), 0)
    oht = ((iota == m) | (iota == w + 12) | (iota == d + 19)).astype(jnp.float32)
    feat = jnp.concatenate([xct, oht], axis=0)
    acc = jax.lax.dot_general(
        feat, wall_ref[...], (((0,), (0,)), ((), ())),
        preferred_element_type=jnp.float32,
    )
    out_ref[...] = acc + peb_ref[...]


def kernel(x_cont, x_month, x_weekday, x_dir, W_proj, b_proj, E_month, E_weekday, E_dir):
    B, L, C = x_cont.shape
    xc2 = x_cont.transpose(2, 1, 0)
    combo = (
        x_month.astype(jnp.int32)
        | (x_weekday.astype(jnp.int32) << 4)
        | (x_dir.astype(jnp.int32) << 8)
    ).T
    w_all = jnp.concatenate([W_proj, E_month, E_weekday, E_dir], axis=0)
    pe_block = jnp.asarray(_PE50)[:, None, :] + b_proj[None, None, :]

    grid = (B // BM,)
    out2 = pl.pallas_call(
        _fused_kernel,
        grid=grid,
        in_specs=[
            pl.BlockSpec((C, L_SEQ, BM), lambda i: (0, 0, i)),
            pl.BlockSpec((L_SEQ, BM), lambda i: (0, i)),
            pl.BlockSpec((C + 27, D_MODEL), lambda i: (0, 0)),
            pl.BlockSpec((L_SEQ, 1, D_MODEL), lambda i: (0, 0, 0)),
        ],
        out_specs=pl.BlockSpec((L_SEQ, BM, D_MODEL), lambda i: (0, i, 0)),
        out_shape=jax.ShapeDtypeStruct((L, B, D_MODEL), jnp.float32),
        compiler_params=pltpu.CompilerParams(
            dimension_semantics=("parallel",),
        ),
    )(xc2, combo, w_all, pe_block)
    return out2.transpose(1, 0, 2)


# speedup vs baseline: 52.4418x; 1.1674x over previous
import math

import jax
import jax.numpy as jnp
import numpy as np
from jax.experimental import pallas as pl
from jax.experimental.pallas import tpu as pltpu

D_MODEL = 128
L_SEQ = 50
BM = 256
R = BM * L_SEQ


def _pe_const(d_model: int, max_len: int) -> np.ndarray:
    pos = np.arange(0, max_len, dtype=np.float32)[:, None]
    div = np.exp(
        np.arange(0, d_model, 2, dtype=np.float32) * (-math.log(10000.0) / d_model)
    )
    pe = np.zeros((max_len, d_model), dtype=np.float32)
    pe[:, 0::2] = np.sin(pos * div)
    pe[:, 1::2] = np.cos(pos * div)
    return pe


_PE50 = _pe_const(D_MODEL, L_SEQ)


def _fused_kernel(xc_ref, combo_ref, wall_ref, peb_ref, out_ref):
    xct = xc_ref[...]
    combo = combo_ref[...][None]
    m = combo & 15
    w = (combo >> 4) & 15
    d = combo >> 8
    iota = jax.lax.broadcasted_iota(jnp.int32, (27, L_SEQ, BM), 0)
    oht = ((iota == m) | (iota == w + 12) | (iota == d + 19)).astype(jnp.float32)
    feat = jnp.concatenate([xct, oht], axis=0)
    acc = jax.lax.dot_general(
        feat, wall_ref[...], (((0,), (0,)), ((), ())),
        preferred_element_type=jnp.float32,
    )
    out_ref[...] = acc + peb_ref[...]


def kernel(x_cont, x_month, x_weekday, x_dir, W_proj, b_proj, E_month, E_weekday, E_dir):
    B, L, C = x_cont.shape
    xc2 = x_cont.transpose(2, 1, 0)
    combo = (
        x_month.astype(jnp.int32)
        | (x_weekday.astype(jnp.int32) << 4)
        | (x_dir.astype(jnp.int32) << 8)
    ).T
    w_all = jnp.concatenate([W_proj, E_month, E_weekday, E_dir], axis=0)
    pe_block = jnp.asarray(_PE50)[:, None, :] + b_proj[None, None, :]

    grid = (B // BM,)
    out2 = pl.pallas_call(
        _fused_kernel,
        grid=grid,
        in_specs=[
            pl.BlockSpec((C, L_SEQ, BM), lambda i: (0, 0, i)),
            pl.BlockSpec((L_SEQ, BM), lambda i: (0, i)),
            pl.BlockSpec((C + 27, D_MODEL), lambda i: (0, 0)),
            pl.BlockSpec((L_SEQ, 1, D_MODEL), lambda i: (0, 0, 0)),
        ],
        out_specs=pl.BlockSpec((L_SEQ, BM, D_MODEL), lambda i: (0, i, 0)),
        out_shape=jax.ShapeDtypeStruct((L, B, D_MODEL), jnp.float32),
        compiler_params=pltpu.CompilerParams(
            dimension_semantics=("parallel",),
        ),
    )(xc2, combo, w_all, pe_block)
    return out2.transpose(1, 0, 2)


# speedup vs baseline: 54.3157x; 1.0357x over previous
import math

import jax
import jax.numpy as jnp
import numpy as np
from jax.experimental import pallas as pl
from jax.experimental.pallas import tpu as pltpu

D_MODEL = 128
L_SEQ = 50
BM = 512
R = BM * L_SEQ


def _pe_const(d_model: int, max_len: int) -> np.ndarray:
    pos = np.arange(0, max_len, dtype=np.float32)[:, None]
    div = np.exp(
        np.arange(0, d_model, 2, dtype=np.float32) * (-math.log(10000.0) / d_model)
    )
    pe = np.zeros((max_len, d_model), dtype=np.float32)
    pe[:, 0::2] = np.sin(pos * div)
    pe[:, 1::2] = np.cos(pos * div)
    return pe


_PE50 = _pe_const(D_MODEL, L_SEQ)


def _fused_kernel(xc_ref, combo_ref, wall_ref, peb_ref, out_ref):
    xct = xc_ref[...]
    combo = combo_ref[...][None]
    m = combo & 15
    w = (combo >> 4) & 15
    d = combo >> 8
    iota = jax.lax.broadcasted_iota(jnp.int32, (27, L_SEQ, BM), 0)
    oht = ((iota == m) | (iota == w + 12) | (iota == d + 19)).astype(jnp.float32)
    feat = jnp.concatenate([xct, oht], axis=0)
    acc = jax.lax.dot_general(
        feat, wall_ref[...], (((0,), (0,)), ((), ())),
        preferred_element_type=jnp.float32,
    )
    out_ref[...] = acc + peb_ref[...]


def kernel(x_cont, x_month, x_weekday, x_dir, W_proj, b_proj, E_month, E_weekday, E_dir):
    B, L, C = x_cont.shape
    xc2 = x_cont.transpose(2, 1, 0)
    combo = (
        x_month.astype(jnp.int32)
        | (x_weekday.astype(jnp.int32) << 4)
        | (x_dir.astype(jnp.int32) << 8)
    ).T
    w_all = jnp.concatenate([W_proj, E_month, E_weekday, E_dir], axis=0)
    pe_block = jnp.asarray(_PE50)[:, None, :] + b_proj[None, None, :]

    grid = (B // BM,)
    out2 = pl.pallas_call(
        _fused_kernel,
        grid=grid,
        in_specs=[
            pl.BlockSpec((C, L_SEQ, BM), lambda i: (0, 0, i)),
            pl.BlockSpec((L_SEQ, BM), lambda i: (0, i)),
            pl.BlockSpec((C + 27, D_MODEL), lambda i: (0, 0)),
            pl.BlockSpec((L_SEQ, 1, D_MODEL), lambda i: (0, 0, 0)),
        ],
        out_specs=pl.BlockSpec((L_SEQ, BM, D_MODEL), lambda i: (0, i, 0)),
        out_shape=jax.ShapeDtypeStruct((L, B, D_MODEL), jnp.float32),
        compiler_params=pltpu.CompilerParams(
            dimension_semantics=("parallel",),
        ),
    )(xc2, combo, w_all, pe_block)
    return out2.transpose(1, 0, 2)
